# baseline (speedup 1.0000x reference)
import functools

import jax
import jax.numpy as jnp
from jax import lax
from jax.experimental import pallas as pl
from jax.experimental.pallas import tpu as pltpu
from jax.experimental.pallas import tpu_sc as plsc

NUM_EMB = 1000000
DIM = 64
PDIM = 2 * DIM
B, F = 16384, 26
FLAT = B * F
NC, NS = 2, 16
NW = NC * NS
PER_W = FLAT // NW
CHUNK = 128
NCHUNK = PER_W // CHUNK
NBUF = 2

_mesh = plsc.VectorSubcoreMesh(core_axis_name="c", subcore_axis_name="s")


@functools.partial(
    pl.kernel,
    out_type=jax.ShapeDtypeStruct((FLAT, PDIM), jnp.float32),
    mesh=_mesh,
    scratch_types=[
        pltpu.VMEM((NCHUNK, CHUNK), jnp.int32),
        pltpu.VMEM((NBUF, CHUNK, PDIM), jnp.float32),
        pltpu.SemaphoreType.DMA((NBUF,)),
        pltpu.SemaphoreType.DMA((NBUF,)),
    ],
    compiler_params=pltpu.CompilerParams(use_tc_tiling_on_sc=True),
)
def _gather_kernel(idx_hbm, table_hbm, out_hbm, idx_v, rows_v, gsem, ssem):
    wid = lax.axis_index("s") * NC + lax.axis_index("c")
    base = wid * PER_W

    pltpu.sync_copy(idx_hbm.at[wid], idx_v)

    def gather_start(chunk, buf):
        pltpu.async_copy(table_hbm.at[idx_v.at[chunk]], rows_v.at[buf],
                         gsem.at[buf])

    def gather_wait(chunk, buf):
        pltpu.make_async_copy(table_hbm.at[idx_v.at[chunk]], rows_v.at[buf],
                              gsem.at[buf]).wait()

    def store_start(chunk, buf):
        pltpu.async_copy(rows_v.at[buf],
                         out_hbm.at[pl.ds(base + chunk * CHUNK, CHUNK)],
                         ssem.at[buf])

    def store_wait(chunk, buf):
        pltpu.make_async_copy(rows_v.at[buf],
                              out_hbm.at[pl.ds(base + chunk * CHUNK, CHUNK)],
                              ssem.at[buf]).wait()

    for b in range(NBUF):
        gather_start(b, b)

    @pl.loop(0, NCHUNK, step=NBUF)
    def _grp(g):
        for b in range(NBUF):
            chunk = g + b
            gather_wait(chunk, b)
            store_start(chunk, b)
            nxt = chunk + NBUF

            @pl.when(nxt < NCHUNK)
            def _():
                store_wait(chunk, b)
                gather_start(nxt, b)

    for b in range(NBUF):
        store_wait(NCHUNK - NBUF + b, b)


def kernel(indices, table):
    flat = indices.reshape(-1).astype(jnp.int32)
    pairs = (flat >> 1).reshape(NW, NCHUNK, CHUNK)
    table2 = table.reshape(NUM_EMB // 2, PDIM)
    outp = _gather_kernel(pairs, table2)
    odd = (flat & 1).astype(bool)[:, None]
    out = jnp.where(odd, outp[:, DIM:], outp[:, :DIM])
    return out.reshape(B, F, DIM)
